# baseline (speedup 1.0000x reference)
import functools

import jax
import jax.numpy as jnp
from jax import lax
from jax.experimental import pallas as pl
from jax.experimental.pallas import tpu as pltpu
from jax.experimental.pallas import tpu_sc as plsc

S = 64.0
M = 0.4
A = 1.3
K = 0.1
B = 1024
C = 100000

_CHUNK = 128


def _sc_gather_chunks(cosine, label):
    info = plsc.get_sparse_core_info()
    nc, ns = info.num_cores, info.num_subcores
    nw = nc * ns
    bpw = B // nw

    mesh = plsc.VectorSubcoreMesh(core_axis_name="c", subcore_axis_name="s")

    @functools.partial(
        pl.kernel,
        out_type=jax.ShapeDtypeStruct((B, 8, _CHUNK), jnp.float32),
        mesh=mesh,
        scratch_types=[
            pltpu.VMEM((bpw,), jnp.int32),
            pltpu.SemaphoreType.DMA,
            pltpu.SemaphoreType.DMA,
        ],
    )
    def k(cos_hbm, label_hbm, tgt_hbm, lab_s, lsem, sem):
        wid = lax.axis_index("s") * nc + lax.axis_index("c")
        base = wid * bpw
        pltpu.async_copy(label_hbm.at[pl.ds(base, bpw)], lab_s, lsem).wait()
        lab_vecs = [lab_s[pl.ds(g * 16, 16)] for g in range(bpw // 16)]
        for r in range(bpw):
            lab = lab_vecs[r // 16][r % 16]
            safe = jnp.where(lab == -1, 0, lab)
            col = pl.multiple_of(jnp.bitwise_and(safe, ~(_CHUNK - 1)), _CHUNK)
            rb = pl.multiple_of((base + r) & ~7, 8)
            pltpu.async_copy(
                cos_hbm.at[pl.ds(rb, 8), pl.ds(col, _CHUNK)],
                tgt_hbm.at[base + r], sem)
        for r in range(bpw):
            pltpu.make_async_copy(
                cos_hbm.at[pl.ds(0, 8), pl.ds(0, _CHUNK)],
                tgt_hbm.at[base + r], sem).wait()

    return k(cosine, label)


_BR = 256
_BC = 2048


def _arccos(t):
    ax = jnp.abs(t)
    p = jnp.float32(-0.0012624911)
    for coef in (0.0066700901, -0.0170881256, 0.0308918810, -0.0501743046,
                 0.0889789874, -0.2145988016, 1.5707963050):
        p = p * ax + jnp.float32(coef)
    r = jnp.sqrt(jnp.maximum(1.0 - ax, 0.0)) * p
    return jnp.where(t >= 0, r, jnp.float32(3.14159265358979) - r)


def _tc_body(lab_ref, chunk_ref, x_ref, o_ref, adj_s):
    i = pl.program_id(0)
    j = pl.program_id(1)

    @pl.when(j == 0)
    def _extract():
        ch = chunk_ref[...]
        lab3 = lab_ref[...][:, :, None]
        safe = jnp.where(lab3 == -1, 0, lab3)
        riota = lax.broadcasted_iota(jnp.int32, (_BR, 8, _CHUNK), 0)
        siota = lax.broadcasted_iota(jnp.int32, (_BR, 8, _CHUNK), 1)
        liota = lax.broadcasted_iota(jnp.int32, (_BR, 8, _CHUNK), 2)
        mask = (siota == jnp.bitwise_and(riota, 7)) & (
            liota == jnp.bitwise_and(safe, _CHUNK - 1))
        t = jnp.sum(jnp.where(mask, ch, 0.0), axis=(1, 2))[:, None]
        theta = _arccos(jnp.clip(t, -1.0, 1.0))
        adj_s[...] = jnp.where(lab_ref[...] != -1,
                               (M + K * A) - K * theta, 0.0) * S

    x = x_ref[...]
    c = jnp.clip(x, -1.0, 1.0)
    lab = lab_ref[...]
    col = j * _BC + lax.broadcasted_iota(jnp.int32, (_BR, _BC), 1)
    o_ref[...] = c * S - jnp.where(col == lab, adj_s[...], 0.0)


def kernel(cosine, label):
    label = label.astype(jnp.int32)
    tchunk = _sc_gather_chunks(cosine, label)

    grid = (B // _BR, pl.cdiv(C, _BC))
    out = pl.pallas_call(
        _tc_body,
        grid=grid,
        in_specs=[
            pl.BlockSpec((_BR, 1), lambda i, j: (i, 0)),
            pl.BlockSpec((_BR, 8, _CHUNK), lambda i, j: (i, 0, 0)),
            pl.BlockSpec((_BR, _BC), lambda i, j: (i, j)),
        ],
        out_specs=pl.BlockSpec((_BR, _BC), lambda i, j: (i, j)),
        out_shape=jax.ShapeDtypeStruct((B, C), jnp.float32),
        scratch_shapes=[pltpu.VMEM((_BR, 1), jnp.float32)],
    )(label[:, None], tchunk, cosine)
    return out


# speedup vs baseline: 1.1302x; 1.1302x over previous
import functools

import jax
import jax.numpy as jnp
from jax import lax
from jax.experimental import pallas as pl
from jax.experimental.pallas import tpu as pltpu
from jax.experimental.pallas import tpu_sc as plsc

S = 64.0
M = 0.4
A = 1.3
K = 0.1
B = 1024
C = 100000

_CHUNK = 128


def _sc_gather_chunks(cosine, label):
    info = plsc.get_sparse_core_info()
    nc, ns = info.num_cores, info.num_subcores
    nw = nc * ns
    bpw = B // nw

    mesh = plsc.VectorSubcoreMesh(core_axis_name="c", subcore_axis_name="s")

    @functools.partial(
        pl.kernel,
        out_type=jax.ShapeDtypeStruct((B, _CHUNK), jnp.float32),
        mesh=mesh,
        scratch_types=[
            pltpu.VMEM((bpw,), jnp.int32),
            pltpu.VMEM((bpw, 8, _CHUNK), jnp.float32),
            pltpu.VMEM((bpw, _CHUNK), jnp.float32),
            pltpu.SemaphoreType.DMA,
            pltpu.SemaphoreType.DMA,
        ],
    )
    def k(cos_hbm, label_hbm, tgt_hbm, lab_s, tiles_v, rows_v, lsem, sem):
        wid = lax.axis_index("s") * nc + lax.axis_index("c")
        base = wid * bpw
        pltpu.async_copy(label_hbm.at[pl.ds(base, bpw)], lab_s, lsem).wait()
        lab_vecs = [lab_s[pl.ds(g * 16, 16)] for g in range(bpw // 16)]
        for r in range(bpw):
            lab = lab_vecs[r // 16][r % 16]
            safe = jnp.where(lab == -1, 0, lab)
            col = pl.multiple_of(jnp.bitwise_and(safe, ~(_CHUNK - 1)), _CHUNK)
            rb = pl.multiple_of((base + r) & ~7, 8)
            pltpu.async_copy(
                cos_hbm.at[pl.ds(rb, 8), pl.ds(col, _CHUNK)],
                tiles_v.at[r], sem)
        for r in range(bpw):
            pltpu.make_async_copy(
                cos_hbm.at[pl.ds(0, 8), pl.ds(0, _CHUNK)],
                tiles_v.at[r], sem).wait()
        for r in range(bpw):
            p = (base + r) & 7
            for q in range(_CHUNK // 16):
                rows_v[r, pl.ds(q * 16, 16)] = tiles_v[r, p, pl.ds(q * 16, 16)]
        pltpu.sync_copy(rows_v, tgt_hbm.at[pl.ds(base, bpw)])

    return k(cosine, label)


_BR = 256
_BC = 2048


def _arccos(t):
    ax = jnp.abs(t)
    p = jnp.float32(-0.0012624911)
    for coef in (0.0066700901, -0.0170881256, 0.0308918810, -0.0501743046,
                 0.0889789874, -0.2145988016, 1.5707963050):
        p = p * ax + jnp.float32(coef)
    r = jnp.sqrt(jnp.maximum(1.0 - ax, 0.0)) * p
    return jnp.where(t >= 0, r, jnp.float32(3.14159265358979) - r)


def _tc_body(lab_ref, chunk_ref, x_ref, o_ref, adj_s):
    j = pl.program_id(1)

    @pl.when(j == 0)
    def _extract():
        lab = lab_ref[...]
        safe = jnp.where(lab == -1, 0, lab)
        off = jnp.bitwise_and(safe, _CHUNK - 1)
        lane = lax.broadcasted_iota(jnp.int32, (_BR, _CHUNK), 1)
        t = jnp.sum(jnp.where(lane == off, chunk_ref[...], 0.0), axis=1,
                    keepdims=True)
        theta = _arccos(jnp.clip(t, -1.0, 1.0))
        adj_s[...] = jnp.where(lab != -1,
                               (M + K * A) - K * theta, 0.0) * S

    x = x_ref[...]
    c = jnp.clip(x, -1.0, 1.0)
    lab = lab_ref[...]
    col = j * _BC + lax.broadcasted_iota(jnp.int32, (_BR, _BC), 1)
    o_ref[...] = c * S - jnp.where(col == lab, adj_s[...], 0.0)


def kernel(cosine, label):
    label = label.astype(jnp.int32)
    tchunk = _sc_gather_chunks(cosine, label)

    grid = (B // _BR, pl.cdiv(C, _BC))
    out = pl.pallas_call(
        _tc_body,
        grid=grid,
        in_specs=[
            pl.BlockSpec((_BR, 1), lambda i, j: (i, 0)),
            pl.BlockSpec((_BR, _CHUNK), lambda i, j: (i, 0)),
            pl.BlockSpec((_BR, _BC), lambda i, j: (i, j)),
        ],
        out_specs=pl.BlockSpec((_BR, _BC), lambda i, j: (i, j)),
        out_shape=jax.ShapeDtypeStruct((B, C), jnp.float32),
        scratch_shapes=[pltpu.VMEM((_BR, 1), jnp.float32)],
    )(label[:, None], tchunk, cosine)
    return out


# speedup vs baseline: 1.1710x; 1.0361x over previous
import functools

import jax
import jax.numpy as jnp
from jax import lax
from jax.experimental import pallas as pl
from jax.experimental.pallas import tpu as pltpu
from jax.experimental.pallas import tpu_sc as plsc

S = 64.0
M = 0.4
A = 1.3
K = 0.1
B = 1024
C = 100000

_CHUNK = 128


def _sc_gather_chunks(cosine, label):
    info = plsc.get_sparse_core_info()
    nc, ns = info.num_cores, info.num_subcores
    nw = nc * ns
    bpw = B // nw

    mesh = plsc.VectorSubcoreMesh(core_axis_name="c", subcore_axis_name="s")

    @functools.partial(
        pl.kernel,
        out_type=jax.ShapeDtypeStruct((B, _CHUNK), jnp.float32),
        mesh=mesh,
        scratch_types=[
            pltpu.VMEM((bpw,), jnp.int32),
            pltpu.VMEM((bpw, 8, _CHUNK), jnp.float32),
            pltpu.VMEM((bpw, _CHUNK), jnp.float32),
            pltpu.SemaphoreType.DMA,
            pltpu.SemaphoreType.DMA,
        ],
    )
    def k(cos_hbm, label_hbm, tgt_hbm, lab_s, tiles_v, rows_v, lsem, sem):
        wid = lax.axis_index("s") * nc + lax.axis_index("c")
        base = wid * bpw
        pltpu.async_copy(label_hbm.at[pl.ds(base, bpw)], lab_s, lsem).wait()
        lab_vecs = [lab_s[pl.ds(g * 16, 16)] for g in range(bpw // 16)]
        for r in range(bpw):
            lab = lab_vecs[r // 16][r % 16]
            safe = jnp.where(lab == -1, 0, lab)
            col = pl.multiple_of(jnp.bitwise_and(safe, ~(_CHUNK - 1)), _CHUNK)
            rb = pl.multiple_of((base + r) & ~7, 8)
            pltpu.async_copy(
                cos_hbm.at[pl.ds(rb, 8), pl.ds(col, _CHUNK)],
                tiles_v.at[r], sem)
        for r in range(bpw):
            pltpu.make_async_copy(
                cos_hbm.at[pl.ds(0, 8), pl.ds(0, _CHUNK)],
                tiles_v.at[r], sem).wait()
        for r in range(bpw):
            p = (base + r) & 7
            for q in range(_CHUNK // 16):
                rows_v[r, pl.ds(q * 16, 16)] = tiles_v[r, p, pl.ds(q * 16, 16)]
        pltpu.sync_copy(rows_v, tgt_hbm.at[pl.ds(base, bpw)])

    return k(cosine, label)


_BR = 512
_BC = 2048


def _arccos(t):
    ax = jnp.abs(t)
    p = jnp.float32(-0.0012624911)
    for coef in (0.0066700901, -0.0170881256, 0.0308918810, -0.0501743046,
                 0.0889789874, -0.2145988016, 1.5707963050):
        p = p * ax + jnp.float32(coef)
    r = jnp.sqrt(jnp.maximum(1.0 - ax, 0.0)) * p
    return jnp.where(t >= 0, r, jnp.float32(3.14159265358979) - r)


def _tc_body(lab_ref, chunk_ref, x_ref, o_ref, adj_s):
    j = pl.program_id(1)

    @pl.when(j == 0)
    def _extract():
        lab = lab_ref[...]
        safe = jnp.where(lab == -1, 0, lab)
        off = jnp.bitwise_and(safe, _CHUNK - 1)
        lane = lax.broadcasted_iota(jnp.int32, (_BR, _CHUNK), 1)
        t = jnp.sum(jnp.where(lane == off, chunk_ref[...], 0.0), axis=1,
                    keepdims=True)
        theta = _arccos(jnp.clip(t, -1.0, 1.0))
        adj_s[...] = jnp.where(lab != -1,
                               (M + K * A) - K * theta, 0.0) * S

    x = x_ref[...]
    c = jnp.clip(x, -1.0, 1.0)
    lab = lab_ref[...]
    col = j * _BC + lax.broadcasted_iota(jnp.int32, (_BR, _BC), 1)
    o_ref[...] = c * S - jnp.where(col == lab, adj_s[...], 0.0)


def kernel(cosine, label):
    label = label.astype(jnp.int32)
    tchunk = _sc_gather_chunks(cosine, label)

    grid = (B // _BR, pl.cdiv(C, _BC))
    out = pl.pallas_call(
        _tc_body,
        grid=grid,
        in_specs=[
            pl.BlockSpec((_BR, 1), lambda i, j: (i, 0)),
            pl.BlockSpec((_BR, _CHUNK), lambda i, j: (i, 0)),
            pl.BlockSpec((_BR, _BC), lambda i, j: (i, j)),
        ],
        out_specs=pl.BlockSpec((_BR, _BC), lambda i, j: (i, j)),
        out_shape=jax.ShapeDtypeStruct((B, C), jnp.float32),
        scratch_shapes=[pltpu.VMEM((_BR, 1), jnp.float32)],
    )(label[:, None], tchunk, cosine)
    return out


# speedup vs baseline: 1.1733x; 1.0020x over previous
import functools

import jax
import jax.numpy as jnp
from jax import lax
from jax.experimental import pallas as pl
from jax.experimental.pallas import tpu as pltpu
from jax.experimental.pallas import tpu_sc as plsc

S = 64.0
M = 0.4
A = 1.3
K = 0.1
B = 1024
C = 100000

_CHUNK = 128


def _sc_gather_chunks(cosine, label):
    info = plsc.get_sparse_core_info()
    nc, ns = info.num_cores, info.num_subcores
    nw = nc * ns
    bpw = B // nw

    mesh = plsc.VectorSubcoreMesh(core_axis_name="c", subcore_axis_name="s")

    @functools.partial(
        pl.kernel,
        out_type=jax.ShapeDtypeStruct((B, _CHUNK), jnp.float32),
        mesh=mesh,
        scratch_types=[
            pltpu.VMEM((bpw,), jnp.int32),
            pltpu.VMEM((bpw, 8, _CHUNK), jnp.float32),
            pltpu.VMEM((bpw, _CHUNK), jnp.float32),
            pltpu.SemaphoreType.DMA,
            pltpu.SemaphoreType.DMA,
        ],
    )
    def k(cos_hbm, label_hbm, tgt_hbm, lab_s, tiles_v, rows_v, lsem, sem):
        wid = lax.axis_index("s") * nc + lax.axis_index("c")
        base = wid * bpw
        pltpu.async_copy(label_hbm.at[pl.ds(base, bpw)], lab_s, lsem).wait()
        lab_vecs = [lab_s[pl.ds(g * 16, 16)] for g in range(bpw // 16)]
        for r in range(bpw):
            lab = lab_vecs[r // 16][r % 16]
            safe = jnp.where(lab == -1, 0, lab)
            col = pl.multiple_of(jnp.bitwise_and(safe, ~(_CHUNK - 1)), _CHUNK)
            rb = pl.multiple_of((base + r) & ~7, 8)
            pltpu.async_copy(
                cos_hbm.at[pl.ds(rb, 8), pl.ds(col, _CHUNK)],
                tiles_v.at[r], sem)
        for r in range(bpw):
            pltpu.make_async_copy(
                cos_hbm.at[pl.ds(0, 8), pl.ds(0, _CHUNK)],
                tiles_v.at[r], sem).wait()
        for r in range(bpw):
            p = (base + r) & 7
            for q in range(_CHUNK // 16):
                rows_v[r, pl.ds(q * 16, 16)] = tiles_v[r, p, pl.ds(q * 16, 16)]
        pltpu.sync_copy(rows_v, tgt_hbm.at[pl.ds(base, bpw)])

    return k(cosine, label)


_BR = 1024
_BC = 2048


def _arccos(t):
    ax = jnp.abs(t)
    p = jnp.float32(-0.0012624911)
    for coef in (0.0066700901, -0.0170881256, 0.0308918810, -0.0501743046,
                 0.0889789874, -0.2145988016, 1.5707963050):
        p = p * ax + jnp.float32(coef)
    r = jnp.sqrt(jnp.maximum(1.0 - ax, 0.0)) * p
    return jnp.where(t >= 0, r, jnp.float32(3.14159265358979) - r)


def _tc_body(lab_ref, chunk_ref, x_ref, o_ref, adj_s):
    j = pl.program_id(1)

    @pl.when(j == 0)
    def _extract():
        lab = lab_ref[...]
        safe = jnp.where(lab == -1, 0, lab)
        off = jnp.bitwise_and(safe, _CHUNK - 1)
        lane = lax.broadcasted_iota(jnp.int32, (_BR, _CHUNK), 1)
        t = jnp.sum(jnp.where(lane == off, chunk_ref[...], 0.0), axis=1,
                    keepdims=True)
        theta = _arccos(jnp.clip(t, -1.0, 1.0))
        adj_s[...] = jnp.where(lab != -1,
                               (M + K * A) - K * theta, 0.0) * S

    x = x_ref[...]
    c = jnp.clip(x, -1.0, 1.0)
    lab = lab_ref[...]
    col = j * _BC + lax.broadcasted_iota(jnp.int32, (_BR, _BC), 1)
    o_ref[...] = c * S - jnp.where(col == lab, adj_s[...], 0.0)


def kernel(cosine, label):
    label = label.astype(jnp.int32)
    tchunk = _sc_gather_chunks(cosine, label)

    grid = (B // _BR, pl.cdiv(C, _BC))
    out = pl.pallas_call(
        _tc_body,
        grid=grid,
        in_specs=[
            pl.BlockSpec((_BR, 1), lambda i, j: (i, 0)),
            pl.BlockSpec((_BR, _CHUNK), lambda i, j: (i, 0)),
            pl.BlockSpec((_BR, _BC), lambda i, j: (i, j)),
        ],
        out_specs=pl.BlockSpec((_BR, _BC), lambda i, j: (i, j)),
        out_shape=jax.ShapeDtypeStruct((B, C), jnp.float32),
        scratch_shapes=[pltpu.VMEM((_BR, 1), jnp.float32)],
    )(label[:, None], tchunk, cosine)
    return out


# speedup vs baseline: 1.1747x; 1.0012x over previous
import functools

import jax
import jax.numpy as jnp
from jax import lax
from jax.experimental import pallas as pl
from jax.experimental.pallas import tpu as pltpu
from jax.experimental.pallas import tpu_sc as plsc

S = 64.0
M = 0.4
A = 1.3
K = 0.1
B = 1024
C = 100000

_CHUNK = 128


def _sc_gather_chunks(cosine, label):
    info = plsc.get_sparse_core_info()
    nc, ns = info.num_cores, info.num_subcores
    nw = nc * ns
    bpw = B // nw

    mesh = plsc.VectorSubcoreMesh(core_axis_name="c", subcore_axis_name="s")

    @functools.partial(
        pl.kernel,
        out_type=jax.ShapeDtypeStruct((B, _CHUNK), jnp.float32),
        mesh=mesh,
        scratch_types=[
            pltpu.VMEM((bpw,), jnp.int32),
            pltpu.VMEM((bpw, 8, _CHUNK), jnp.float32),
            pltpu.VMEM((bpw, _CHUNK), jnp.float32),
            pltpu.SemaphoreType.DMA,
            pltpu.SemaphoreType.DMA,
        ],
    )
    def k(cos_hbm, label_hbm, tgt_hbm, lab_s, tiles_v, rows_v, lsem, sem):
        wid = lax.axis_index("s") * nc + lax.axis_index("c")
        base = wid * bpw
        pltpu.async_copy(label_hbm.at[pl.ds(base, bpw)], lab_s, lsem).wait()
        lab_vecs = [lab_s[pl.ds(g * 16, 16)] for g in range(bpw // 16)]
        for r in range(bpw):
            lab = lab_vecs[r // 16][r % 16]
            safe = jnp.where(lab == -1, 0, lab)
            col = pl.multiple_of(jnp.bitwise_and(safe, ~(_CHUNK - 1)), _CHUNK)
            rb = pl.multiple_of((base + r) & ~7, 8)
            pltpu.async_copy(
                cos_hbm.at[pl.ds(rb, 8), pl.ds(col, _CHUNK)],
                tiles_v.at[r], sem)
        for r in range(bpw):
            pltpu.make_async_copy(
                cos_hbm.at[pl.ds(0, 8), pl.ds(0, _CHUNK)],
                tiles_v.at[r], sem).wait()
        for r in range(bpw):
            p = (base + r) & 7
            for q in range(_CHUNK // 16):
                rows_v[r, pl.ds(q * 16, 16)] = tiles_v[r, p, pl.ds(q * 16, 16)]
        pltpu.sync_copy(rows_v, tgt_hbm.at[pl.ds(base, bpw)])

    return k(cosine, label)


_BR = 512
_BC = 4096


def _arccos(t):
    ax = jnp.abs(t)
    p = jnp.float32(-0.0012624911)
    for coef in (0.0066700901, -0.0170881256, 0.0308918810, -0.0501743046,
                 0.0889789874, -0.2145988016, 1.5707963050):
        p = p * ax + jnp.float32(coef)
    r = jnp.sqrt(jnp.maximum(1.0 - ax, 0.0)) * p
    return jnp.where(t >= 0, r, jnp.float32(3.14159265358979) - r)


def _tc_body(lab_ref, chunk_ref, x_ref, o_ref, adj_s):
    j = pl.program_id(1)

    @pl.when(j == 0)
    def _extract():
        lab = lab_ref[...]
        safe = jnp.where(lab == -1, 0, lab)
        off = jnp.bitwise_and(safe, _CHUNK - 1)
        lane = lax.broadcasted_iota(jnp.int32, (_BR, _CHUNK), 1)
        t = jnp.sum(jnp.where(lane == off, chunk_ref[...], 0.0), axis=1,
                    keepdims=True)
        theta = _arccos(jnp.clip(t, -1.0, 1.0))
        adj_s[...] = jnp.where(lab != -1,
                               (M + K * A) - K * theta, 0.0) * S

    x = x_ref[...]
    c = jnp.clip(x, -1.0, 1.0)
    lab = lab_ref[...]
    col = j * _BC + lax.broadcasted_iota(jnp.int32, (_BR, _BC), 1)
    o_ref[...] = c * S - jnp.where(col == lab, adj_s[...], 0.0)


def kernel(cosine, label):
    label = label.astype(jnp.int32)
    tchunk = _sc_gather_chunks(cosine, label)

    grid = (B // _BR, pl.cdiv(C, _BC))
    out = pl.pallas_call(
        _tc_body,
        grid=grid,
        in_specs=[
            pl.BlockSpec((_BR, 1), lambda i, j: (i, 0)),
            pl.BlockSpec((_BR, _CHUNK), lambda i, j: (i, 0)),
            pl.BlockSpec((_BR, _BC), lambda i, j: (i, j)),
        ],
        out_specs=pl.BlockSpec((_BR, _BC), lambda i, j: (i, j)),
        out_shape=jax.ShapeDtypeStruct((B, C), jnp.float32),
        scratch_shapes=[pltpu.VMEM((_BR, 1), jnp.float32)],
    )(label[:, None], tchunk, cosine)
    return out
